# baseline (speedup 1.0000x reference)
import functools

import jax
import jax.numpy as jnp
from jax import lax
from jax.experimental import pallas as pl
from jax.experimental.pallas import tpu as pltpu
from jax.experimental.pallas import tpu_sc as plsc

N_CODES = 8192
DIM = 256
ROWS = 8192

ROW_TILE = 512
CODE_TILE = 512
R_GRID = ROWS // ROW_TILE
C_GRID = N_CODES // CODE_TILE


def _argmin_body(x_ref, cb_ref, idx_ref, minval_ref, argmin_ref):
    c = pl.program_id(1)
    x = x_ref[...]
    cb = cb_ref[...]
    scores = lax.dot_general(x.astype(jnp.bfloat16), cb.astype(jnp.bfloat16),
                             (((1,), (1,)), ((), ())),
                             preferred_element_type=jnp.float32)
    x_sq = jnp.sum(x * x, axis=1, keepdims=True)
    e_sq = jnp.sum(cb * cb, axis=1)[None, :]
    dist = x_sq - 2.0 * scores + e_sq
    local_min = jnp.min(dist, axis=1)
    cols = lax.broadcasted_iota(jnp.int32, dist.shape, 1) + c * CODE_TILE
    big = jnp.int32(2**31 - 1)
    local_arg = jnp.min(jnp.where(dist == local_min[:, None], cols, big),
                        axis=1)

    @pl.when(c == 0)
    def _init():
        minval_ref[...] = local_min
        argmin_ref[...] = local_arg

    @pl.when(c > 0)
    def _update():
        cur = minval_ref[...]
        take = local_min < cur
        minval_ref[...] = jnp.where(take, local_min, cur)
        argmin_ref[...] = jnp.where(take, local_arg, argmin_ref[...])

    @pl.when(c == C_GRID - 1)
    def _write():
        idx_ref[0, 0, :] = argmin_ref[...]


def _nearest_idx(flat, codebook):
    idx3 = pl.pallas_call(
        _argmin_body,
        grid=(R_GRID, C_GRID),
        in_specs=[
            pl.BlockSpec((ROW_TILE, DIM), lambda r, c: (r, 0)),
            pl.BlockSpec((CODE_TILE, DIM), lambda r, c: (c, 0)),
        ],
        out_specs=pl.BlockSpec((1, 1, ROW_TILE), lambda r, c: (r, 0, 0)),
        out_shape=jax.ShapeDtypeStruct((R_GRID, 1, ROW_TILE), jnp.int32),
        scratch_shapes=[
            pltpu.VMEM((ROW_TILE,), jnp.float32),
            pltpu.VMEM((ROW_TILE,), jnp.int32),
        ],
        compiler_params=pltpu.CompilerParams(
            dimension_semantics=("arbitrary", "arbitrary")),
    )(flat, codebook)
    return idx3.reshape(ROWS)


def _make_gather():
    info = plsc.get_sparse_core_info()
    nw = info.num_cores * info.num_subcores
    b_per_w = ROWS // nw
    n_chunk = 2
    b_chunk = b_per_w // n_chunk
    mesh = plsc.VectorSubcoreMesh(core_axis_name="c", subcore_axis_name="s")

    @functools.partial(
        pl.kernel, mesh=mesh,
        out_type=jax.ShapeDtypeStruct((ROWS, DIM), jnp.float32),
        scratch_types=[
            pltpu.VMEM((b_chunk,), jnp.int32),
            pltpu.VMEM((b_chunk, DIM), jnp.float32),
            pltpu.SemaphoreType.DMA,
        ],
    )
    def gather_k(table_hbm, idx_hbm, out_hbm, idx_v, rows_v, sem):
        wid = lax.axis_index("s") * info.num_cores + lax.axis_index("c")
        base = wid * b_per_w
        for ch in range(n_chunk):
            off = base + ch * b_chunk
            pltpu.sync_copy(idx_hbm.at[pl.ds(off, b_chunk)], idx_v)
            pltpu.async_copy(table_hbm.at[idx_v], rows_v, sem).wait()
            pltpu.sync_copy(rows_v, out_hbm.at[pl.ds(off, b_chunk)])

    return gather_k


def kernel(x, epoch, codebook):
    b, n, d = x.shape
    flat = x.reshape(-1, d)
    idx = _nearest_idx(flat, codebook)
    quantized = _make_gather()(codebook, idx)
    return quantized.reshape(b, n, d)


# speedup vs baseline: 1.5978x; 1.5978x over previous
import functools

import jax
import jax.numpy as jnp
from jax import lax
from jax.experimental import pallas as pl
from jax.experimental.pallas import tpu as pltpu
from jax.experimental.pallas import tpu_sc as plsc

N_CODES = 8192
DIM = 256
ROWS = 8192

ROW_TILE = 512
CODE_TILE = 1024
R_GRID = ROWS // ROW_TILE
C_GRID = N_CODES // CODE_TILE


def _argmin_body(x_ref, cb_ref, idx_ref, minval_ref, argmin_ref):
    c = pl.program_id(1)
    x = x_ref[...]
    cb = cb_ref[...]
    scores = lax.dot_general(x.astype(jnp.bfloat16), cb.astype(jnp.bfloat16),
                             (((1,), (1,)), ((), ())),
                             preferred_element_type=jnp.float32)
    x_sq = jnp.sum(x * x, axis=1, keepdims=True)
    e_sq = jnp.sum(cb * cb, axis=1)[None, :]
    dist = x_sq - 2.0 * scores + e_sq
    local_min = jnp.min(dist, axis=1)
    cols = lax.broadcasted_iota(jnp.int32, dist.shape, 1) + c * CODE_TILE
    big = jnp.int32(2**31 - 1)
    local_arg = jnp.min(jnp.where(dist == local_min[:, None], cols, big),
                        axis=1)

    @pl.when(c == 0)
    def _init():
        minval_ref[...] = local_min
        argmin_ref[...] = local_arg

    @pl.when(c > 0)
    def _update():
        cur = minval_ref[...]
        take = local_min < cur
        minval_ref[...] = jnp.where(take, local_min, cur)
        argmin_ref[...] = jnp.where(take, local_arg, argmin_ref[...])

    @pl.when(c == C_GRID - 1)
    def _write():
        idx_ref[0, 0, :] = argmin_ref[...]


def _nearest_idx(flat, codebook):
    idx3 = pl.pallas_call(
        _argmin_body,
        grid=(R_GRID, C_GRID),
        in_specs=[
            pl.BlockSpec((ROW_TILE, DIM), lambda r, c: (r, 0)),
            pl.BlockSpec((CODE_TILE, DIM), lambda r, c: (c, 0)),
        ],
        out_specs=pl.BlockSpec((1, 1, ROW_TILE), lambda r, c: (r, 0, 0)),
        out_shape=jax.ShapeDtypeStruct((R_GRID, 1, ROW_TILE), jnp.int32),
        scratch_shapes=[
            pltpu.VMEM((ROW_TILE,), jnp.float32),
            pltpu.VMEM((ROW_TILE,), jnp.int32),
        ],
        compiler_params=pltpu.CompilerParams(
            dimension_semantics=("arbitrary", "arbitrary")),
    )(flat, codebook)
    return idx3.reshape(ROWS)


def _make_gather():
    info = plsc.get_sparse_core_info()
    nw = info.num_cores * info.num_subcores
    b_per_w = ROWS // nw
    n_chunk = 2
    b_chunk = b_per_w // n_chunk
    mesh = plsc.VectorSubcoreMesh(core_axis_name="c", subcore_axis_name="s")

    @functools.partial(
        pl.kernel, mesh=mesh,
        out_type=jax.ShapeDtypeStruct((ROWS, DIM), jnp.float32),
        scratch_types=[
            pltpu.VMEM((b_chunk,), jnp.int32),
            pltpu.VMEM((b_chunk, DIM), jnp.float32),
            pltpu.SemaphoreType.DMA,
        ],
    )
    def gather_k(table_hbm, idx_hbm, out_hbm, idx_v, rows_v, sem):
        wid = lax.axis_index("s") * info.num_cores + lax.axis_index("c")
        base = wid * b_per_w
        for ch in range(n_chunk):
            off = base + ch * b_chunk
            pltpu.sync_copy(idx_hbm.at[pl.ds(off, b_chunk)], idx_v)
            pltpu.async_copy(table_hbm.at[idx_v], rows_v, sem).wait()
            pltpu.sync_copy(rows_v, out_hbm.at[pl.ds(off, b_chunk)])

    return gather_k


def kernel(x, epoch, codebook):
    b, n, d = x.shape
    flat = x.reshape(-1, d)
    idx = _nearest_idx(flat, codebook)
    quantized = _make_gather()(codebook, idx)
    return quantized.reshape(b, n, d)


# speedup vs baseline: 2.1167x; 1.3247x over previous
import functools

import jax
import jax.numpy as jnp
from jax import lax
from jax.experimental import pallas as pl
from jax.experimental.pallas import tpu as pltpu
from jax.experimental.pallas import tpu_sc as plsc

N_CODES = 8192
DIM = 256
ROWS = 8192

ROW_TILE = 512
CODE_TILE = 2048
R_GRID = ROWS // ROW_TILE
C_GRID = N_CODES // CODE_TILE


def _argmin_body(x_ref, cb_ref, idx_ref, minval_ref, argmin_ref):
    c = pl.program_id(1)
    x = x_ref[...]
    cb = cb_ref[...]
    scores = lax.dot_general(x.astype(jnp.bfloat16), cb.astype(jnp.bfloat16),
                             (((1,), (1,)), ((), ())),
                             preferred_element_type=jnp.float32)
    x_sq = jnp.sum(x * x, axis=1, keepdims=True)
    e_sq = jnp.sum(cb * cb, axis=1)[None, :]
    dist = x_sq - 2.0 * scores + e_sq
    local_min = jnp.min(dist, axis=1)
    cols = lax.broadcasted_iota(jnp.int32, dist.shape, 1) + c * CODE_TILE
    big = jnp.int32(2**31 - 1)
    local_arg = jnp.min(jnp.where(dist == local_min[:, None], cols, big),
                        axis=1)

    @pl.when(c == 0)
    def _init():
        minval_ref[...] = local_min
        argmin_ref[...] = local_arg

    @pl.when(c > 0)
    def _update():
        cur = minval_ref[...]
        take = local_min < cur
        minval_ref[...] = jnp.where(take, local_min, cur)
        argmin_ref[...] = jnp.where(take, local_arg, argmin_ref[...])

    @pl.when(c == C_GRID - 1)
    def _write():
        idx_ref[0, 0, :] = argmin_ref[...]


def _nearest_idx(flat, codebook):
    idx3 = pl.pallas_call(
        _argmin_body,
        grid=(R_GRID, C_GRID),
        in_specs=[
            pl.BlockSpec((ROW_TILE, DIM), lambda r, c: (r, 0)),
            pl.BlockSpec((CODE_TILE, DIM), lambda r, c: (c, 0)),
        ],
        out_specs=pl.BlockSpec((1, 1, ROW_TILE), lambda r, c: (r, 0, 0)),
        out_shape=jax.ShapeDtypeStruct((R_GRID, 1, ROW_TILE), jnp.int32),
        scratch_shapes=[
            pltpu.VMEM((ROW_TILE,), jnp.float32),
            pltpu.VMEM((ROW_TILE,), jnp.int32),
        ],
        compiler_params=pltpu.CompilerParams(
            dimension_semantics=("arbitrary", "arbitrary")),
    )(flat, codebook)
    return idx3.reshape(ROWS)


def _make_gather():
    info = plsc.get_sparse_core_info()
    nw = info.num_cores * info.num_subcores
    b_per_w = ROWS // nw
    n_chunk = 2
    b_chunk = b_per_w // n_chunk
    mesh = plsc.VectorSubcoreMesh(core_axis_name="c", subcore_axis_name="s")

    @functools.partial(
        pl.kernel, mesh=mesh,
        out_type=jax.ShapeDtypeStruct((ROWS, DIM), jnp.float32),
        scratch_types=[
            pltpu.VMEM((b_chunk,), jnp.int32),
            pltpu.VMEM((b_chunk, DIM), jnp.float32),
            pltpu.SemaphoreType.DMA,
        ],
    )
    def gather_k(table_hbm, idx_hbm, out_hbm, idx_v, rows_v, sem):
        wid = lax.axis_index("s") * info.num_cores + lax.axis_index("c")
        base = wid * b_per_w
        for ch in range(n_chunk):
            off = base + ch * b_chunk
            pltpu.sync_copy(idx_hbm.at[pl.ds(off, b_chunk)], idx_v)
            pltpu.async_copy(table_hbm.at[idx_v], rows_v, sem).wait()
            pltpu.sync_copy(rows_v, out_hbm.at[pl.ds(off, b_chunk)])

    return gather_k


def kernel(x, epoch, codebook):
    b, n, d = x.shape
    flat = x.reshape(-1, d)
    idx = _nearest_idx(flat, codebook)
    quantized = _make_gather()(codebook, idx)
    return quantized.reshape(b, n, d)


# speedup vs baseline: 2.5176x; 1.1894x over previous
import functools

import jax
import jax.numpy as jnp
from jax import lax
from jax.experimental import pallas as pl
from jax.experimental.pallas import tpu as pltpu
from jax.experimental.pallas import tpu_sc as plsc

N_CODES = 8192
DIM = 256
ROWS = 8192

ROW_TILE = 512
CODE_TILE = 4096
R_GRID = ROWS // ROW_TILE
C_GRID = N_CODES // CODE_TILE


def _argmin_body(x_ref, cb_ref, idx_ref, minval_ref, argmin_ref):
    c = pl.program_id(1)
    x = x_ref[...]
    cb = cb_ref[...]
    scores = lax.dot_general(x.astype(jnp.bfloat16), cb.astype(jnp.bfloat16),
                             (((1,), (1,)), ((), ())),
                             preferred_element_type=jnp.float32)
    x_sq = jnp.sum(x * x, axis=1, keepdims=True)
    e_sq = jnp.sum(cb * cb, axis=1)[None, :]
    dist = x_sq - 2.0 * scores + e_sq
    local_min = jnp.min(dist, axis=1)
    cols = lax.broadcasted_iota(jnp.int32, dist.shape, 1) + c * CODE_TILE
    big = jnp.int32(2**31 - 1)
    local_arg = jnp.min(jnp.where(dist == local_min[:, None], cols, big),
                        axis=1)

    @pl.when(c == 0)
    def _init():
        minval_ref[...] = local_min
        argmin_ref[...] = local_arg

    @pl.when(c > 0)
    def _update():
        cur = minval_ref[...]
        take = local_min < cur
        minval_ref[...] = jnp.where(take, local_min, cur)
        argmin_ref[...] = jnp.where(take, local_arg, argmin_ref[...])

    @pl.when(c == C_GRID - 1)
    def _write():
        idx_ref[0, 0, :] = argmin_ref[...]


def _nearest_idx(flat, codebook):
    idx3 = pl.pallas_call(
        _argmin_body,
        grid=(R_GRID, C_GRID),
        in_specs=[
            pl.BlockSpec((ROW_TILE, DIM), lambda r, c: (r, 0)),
            pl.BlockSpec((CODE_TILE, DIM), lambda r, c: (c, 0)),
        ],
        out_specs=pl.BlockSpec((1, 1, ROW_TILE), lambda r, c: (r, 0, 0)),
        out_shape=jax.ShapeDtypeStruct((R_GRID, 1, ROW_TILE), jnp.int32),
        scratch_shapes=[
            pltpu.VMEM((ROW_TILE,), jnp.float32),
            pltpu.VMEM((ROW_TILE,), jnp.int32),
        ],
        compiler_params=pltpu.CompilerParams(
            dimension_semantics=("arbitrary", "arbitrary")),
    )(flat, codebook)
    return idx3.reshape(ROWS)


def _make_gather():
    info = plsc.get_sparse_core_info()
    nw = info.num_cores * info.num_subcores
    b_per_w = ROWS // nw
    n_chunk = 2
    b_chunk = b_per_w // n_chunk
    mesh = plsc.VectorSubcoreMesh(core_axis_name="c", subcore_axis_name="s")

    @functools.partial(
        pl.kernel, mesh=mesh,
        out_type=jax.ShapeDtypeStruct((ROWS, DIM), jnp.float32),
        scratch_types=[
            pltpu.VMEM((b_chunk,), jnp.int32),
            pltpu.VMEM((b_chunk, DIM), jnp.float32),
            pltpu.SemaphoreType.DMA,
        ],
    )
    def gather_k(table_hbm, idx_hbm, out_hbm, idx_v, rows_v, sem):
        wid = lax.axis_index("s") * info.num_cores + lax.axis_index("c")
        base = wid * b_per_w
        for ch in range(n_chunk):
            off = base + ch * b_chunk
            pltpu.sync_copy(idx_hbm.at[pl.ds(off, b_chunk)], idx_v)
            pltpu.async_copy(table_hbm.at[idx_v], rows_v, sem).wait()
            pltpu.sync_copy(rows_v, out_hbm.at[pl.ds(off, b_chunk)])

    return gather_k


def kernel(x, epoch, codebook):
    b, n, d = x.shape
    flat = x.reshape(-1, d)
    idx = _nearest_idx(flat, codebook)
    quantized = _make_gather()(codebook, idx)
    return quantized.reshape(b, n, d)


# speedup vs baseline: 2.7355x; 1.0865x over previous
import functools

import jax
import jax.numpy as jnp
from jax import lax
from jax.experimental import pallas as pl
from jax.experimental.pallas import tpu as pltpu
from jax.experimental.pallas import tpu_sc as plsc

N_CODES = 8192
DIM = 256
ROWS = 8192

ROW_TILE = 512
CODE_TILE = 8192
R_GRID = ROWS // ROW_TILE
C_GRID = N_CODES // CODE_TILE


def _argmin_body(x_ref, cb_ref, idx_ref, minval_ref, argmin_ref):
    c = pl.program_id(1)
    x = x_ref[...]
    cb = cb_ref[...]
    scores = lax.dot_general(x.astype(jnp.bfloat16), cb.astype(jnp.bfloat16),
                             (((1,), (1,)), ((), ())),
                             preferred_element_type=jnp.float32)
    x_sq = jnp.sum(x * x, axis=1, keepdims=True)
    e_sq = jnp.sum(cb * cb, axis=1)[None, :]
    dist = x_sq - 2.0 * scores + e_sq
    local_min = jnp.min(dist, axis=1)
    cols = lax.broadcasted_iota(jnp.int32, dist.shape, 1) + c * CODE_TILE
    big = jnp.int32(2**31 - 1)
    local_arg = jnp.min(jnp.where(dist == local_min[:, None], cols, big),
                        axis=1)

    @pl.when(c == 0)
    def _init():
        minval_ref[...] = local_min
        argmin_ref[...] = local_arg

    @pl.when(c > 0)
    def _update():
        cur = minval_ref[...]
        take = local_min < cur
        minval_ref[...] = jnp.where(take, local_min, cur)
        argmin_ref[...] = jnp.where(take, local_arg, argmin_ref[...])

    @pl.when(c == C_GRID - 1)
    def _write():
        idx_ref[0, 0, :] = argmin_ref[...]


def _nearest_idx(flat, codebook):
    idx3 = pl.pallas_call(
        _argmin_body,
        grid=(R_GRID, C_GRID),
        in_specs=[
            pl.BlockSpec((ROW_TILE, DIM), lambda r, c: (r, 0)),
            pl.BlockSpec((CODE_TILE, DIM), lambda r, c: (c, 0)),
        ],
        out_specs=pl.BlockSpec((1, 1, ROW_TILE), lambda r, c: (r, 0, 0)),
        out_shape=jax.ShapeDtypeStruct((R_GRID, 1, ROW_TILE), jnp.int32),
        scratch_shapes=[
            pltpu.VMEM((ROW_TILE,), jnp.float32),
            pltpu.VMEM((ROW_TILE,), jnp.int32),
        ],
        compiler_params=pltpu.CompilerParams(
            dimension_semantics=("arbitrary", "arbitrary")),
    )(flat, codebook)
    return idx3.reshape(ROWS)


def _make_gather():
    info = plsc.get_sparse_core_info()
    nw = info.num_cores * info.num_subcores
    b_per_w = ROWS // nw
    n_chunk = 2
    b_chunk = b_per_w // n_chunk
    mesh = plsc.VectorSubcoreMesh(core_axis_name="c", subcore_axis_name="s")

    @functools.partial(
        pl.kernel, mesh=mesh,
        out_type=jax.ShapeDtypeStruct((ROWS, DIM), jnp.float32),
        scratch_types=[
            pltpu.VMEM((b_chunk,), jnp.int32),
            pltpu.VMEM((b_chunk, DIM), jnp.float32),
            pltpu.SemaphoreType.DMA,
        ],
    )
    def gather_k(table_hbm, idx_hbm, out_hbm, idx_v, rows_v, sem):
        wid = lax.axis_index("s") * info.num_cores + lax.axis_index("c")
        base = wid * b_per_w
        for ch in range(n_chunk):
            off = base + ch * b_chunk
            pltpu.sync_copy(idx_hbm.at[pl.ds(off, b_chunk)], idx_v)
            pltpu.async_copy(table_hbm.at[idx_v], rows_v, sem).wait()
            pltpu.sync_copy(rows_v, out_hbm.at[pl.ds(off, b_chunk)])

    return gather_k


def kernel(x, epoch, codebook):
    b, n, d = x.shape
    flat = x.reshape(-1, d)
    idx = _nearest_idx(flat, codebook)
    quantized = _make_gather()(codebook, idx)
    return quantized.reshape(b, n, d)
